# baseline (speedup 1.0000x reference)
import functools

import jax
import jax.numpy as jnp
from jax import lax
from jax.experimental import pallas as pl
from jax.experimental.pallas import tpu as pltpu
from jax.experimental.pallas import tpu_sc as plsc

N = 50000
D = 128
E = 160000
R = 3

NC = 2
NS = 16
LANES = 16

EPT = E // NS
CHUNK = 128
NCH = 80
EPAD = NCH * CHUNK

NP = 50176
TPT = NP // NS
ZCH = 784
WPT = N // NS
WCH = 125
ZBR = 64

CB = 4
CW = D // CB


def _zero_fill(buf, rows, cols):
    z = jnp.zeros((LANES,), jnp.float32)

    def body(i, _):
        for k in range(cols // LANES):
            buf[i, pl.ds(k * LANES, LANES)] = z
        return 0

    lax.fori_loop(0, rows, body, 0)


def _deg_body(srcd, dstd, out, idxv, val, wb, acc):
    c = lax.axis_index("c")
    s = lax.axis_index("s")

    _zero_fill(wb, ZCH, 16)
    for q in range(TPT // ZCH):
        pltpu.sync_copy(wb, acc.at[pl.ds(s * TPT + q * ZCH, ZCH)])
    plsc.subcore_barrier()

    lane = lax.iota(jnp.int32, LANES)

    def run_task(t, idx_arr, r):
        oh = jnp.where(lane == t, 1.0, 0.0).astype(jnp.float32)

        def fill(i, _):
            val[i, pl.ds(0, LANES)] = oh
            return 0

        lax.fori_loop(0, CHUNK, fill, 0)
        pltpu.sync_copy(idx_arr.at[r, s], idxv)

        def body(j, _):
            pltpu.sync_copy(val, acc.at[idxv.at[j]], add=True)
            return 0

        lax.fori_loop(0, NCH, body, 0)

    for core in range(NC):
        @pl.when(c == core)
        def _():
            for t in range(3 * core, 3 * core + 3):
                r, which = divmod(t, 2)
                run_task(t, srcd if which == 0 else dstd, r)

    plsc.subcore_barrier()
    for q in range(TPT // ZCH):
        row = s * TPT + q * ZCH
        pltpu.sync_copy(acc.at[pl.ds(row, ZCH)], wb)
        pltpu.sync_copy(wb, out.at[c, pl.ds(row, ZCH)])


def _deg_call(srcd, dstd):
    fn = pl.kernel(
        _deg_body,
        out_type=jax.ShapeDtypeStruct((NC, NP, 16), jnp.float32),
        compiler_params=pltpu.CompilerParams(use_tc_tiling_on_sc=False),
        mesh=plsc.VectorSubcoreMesh(core_axis_name="c", subcore_axis_name="s"),
        scratch_types=[
            pltpu.VMEM((NCH, CHUNK), jnp.int32),
            pltpu.VMEM((CHUNK, 16), jnp.float32),
            pltpu.VMEM((ZCH, 16), jnp.float32),
            pltpu.VMEM_SHARED((NP, 16), jnp.float32),
        ],
    )
    return fn(srcd, dstd)


def _agg_body(hs12, srcg, dstd, out, gidx, didx, gb0, gb1, zb, acc,
              sem0, sem1):
    c = lax.axis_index("c")
    s = lax.axis_index("s")

    _zero_fill(zb, ZBR, CW)

    def run_pass(r, cb):
        def zq(q, _):
            pltpu.sync_copy(zb, acc.at[pl.ds(s * TPT + q * ZBR, ZBR)])
            return 0

        lax.fori_loop(0, TPT // ZBR, zq, 0)
        plsc.subcore_barrier()

        pltpu.sync_copy(srcg.at[r, s], gidx)
        pltpu.sync_copy(dstd.at[r, s], didx)
        base = (r * CB) * N + cb

        def xform(j, _):
            for k in range(CHUNK // LANES):
                v = gidx[j, pl.ds(k * LANES, LANES)]
                gidx[j, pl.ds(k * LANES, LANES)] = v * CB + base
            return 0

        lax.fori_loop(0, NCH, xform, 0)

        pltpu.async_copy(hs12.at[gidx.at[0]], gb0, sem0)

        def pair(jj, _):
            j0 = 2 * jj
            pltpu.async_copy(hs12.at[gidx.at[j0 + 1]], gb1, sem1)
            pltpu.make_async_copy(hs12.at[gidx.at[j0]], gb0, sem0).wait()
            pltpu.sync_copy(gb0, acc.at[didx.at[j0]], add=True)

            @pl.when(jj + 1 < NCH // 2)
            def _():
                pltpu.async_copy(hs12.at[gidx.at[j0 + 2]], gb0, sem0)

            pltpu.make_async_copy(hs12.at[gidx.at[j0 + 1]], gb1, sem1).wait()
            pltpu.sync_copy(gb1, acc.at[didx.at[j0 + 1]], add=True)
            return 0

        lax.fori_loop(0, NCH // 2, pair, 0)
        plsc.subcore_barrier()

        pltpu.sync_copy(acc.at[pl.ds(s * WPT, WPT)],
                        out.at[r, pl.ds(s * WPT, WPT), pl.ds(cb * CW, CW)])
        plsc.subcore_barrier()

    for core in range(NC):
        @pl.when(c == core)
        def _():
            for r in range(R):
                for half in range(CB // NC):
                    run_pass(r, core * (CB // NC) + half)


def _agg_call(hs, srcg, dstd):
    hs12 = hs.reshape(R * N * CB, CW)
    fn = pl.kernel(
        _agg_body,
        out_type=jax.ShapeDtypeStruct((R, N, D), jnp.float32),
        compiler_params=pltpu.CompilerParams(use_tc_tiling_on_sc=False),
        mesh=plsc.VectorSubcoreMesh(core_axis_name="c", subcore_axis_name="s"),
        scratch_types=[
            pltpu.VMEM((NCH, CHUNK), jnp.int32),
            pltpu.VMEM((NCH, CHUNK), jnp.int32),
            pltpu.VMEM((CHUNK, CW), jnp.float32),
            pltpu.VMEM((CHUNK, CW), jnp.float32),
            pltpu.VMEM((ZBR, CW), jnp.float32),
            pltpu.VMEM_SHARED((NP, CW), jnp.float32),
            pltpu.SemaphoreType.DMA,
            pltpu.SemaphoreType.DMA,
        ],
    )
    return fn(hs12, srcg, dstd)


BN = 2000
GRID = N // BN


def _rsqrt_deg(cnt_blk):
    c = cnt_blk[0] + cnt_blk[1]
    return lax.rsqrt(jnp.maximum(c, 1.0))


def _prep_body(x_ref, cnt_ref, hs_ref):
    d = _rsqrt_deg(cnt_ref[...])
    x = x_ref[...]
    for r in range(R):
        hs_ref[r] = x * d[:, 2 * r:2 * r + 1]


def _prep_call(x, cnt):
    return pl.pallas_call(
        _prep_body,
        grid=(GRID,),
        in_specs=[
            pl.BlockSpec((BN, D), lambda i: (i, 0)),
            pl.BlockSpec((NC, BN, 16), lambda i: (0, i, 0)),
        ],
        out_specs=pl.BlockSpec((R, BN, D), lambda i: (0, i, 0)),
        out_shape=jax.ShapeDtypeStruct((R, N, D), jnp.float32),
    )(x, cnt)


def _layer_body(agg_ref, w_ref, b_ref, cnt_ref, out_ref, *, relu_scale):
    d = _rsqrt_deg(cnt_ref[...])
    w = w_ref[...]
    b = b_ref[...]
    acc = (b[0] + b[1] + b[2])[None, :]
    for r in range(R):
        m = jnp.dot(agg_ref[r], w[r], preferred_element_type=jnp.float32)
        acc = acc + m * d[:, 2 * r + 1:2 * r + 2]
    if relu_scale:
        h = jnp.maximum(acc, 0.0)
        for r in range(R):
            out_ref[r] = h * d[:, 2 * r:2 * r + 1]
    else:
        out_ref[...] = acc


def _layer_call(agg, w, b, cnt, relu_scale):
    h = w.shape[-1]
    if relu_scale:
        out_shape = jax.ShapeDtypeStruct((R, N, h), jnp.float32)
        out_spec = pl.BlockSpec((R, BN, h), lambda i: (0, i, 0))
    else:
        out_shape = jax.ShapeDtypeStruct((N, h), jnp.float32)
        out_spec = pl.BlockSpec((BN, h), lambda i: (i, 0))
    return pl.pallas_call(
        functools.partial(_layer_body, relu_scale=relu_scale),
        grid=(GRID,),
        in_specs=[
            pl.BlockSpec((R, BN, D), lambda i: (0, i, 0)),
            pl.BlockSpec((R, D, h), lambda i: (0, 0, 0)),
            pl.BlockSpec((R, h), lambda i: (0, 0)),
            pl.BlockSpec((NC, BN, 16), lambda i: (0, i, 0)),
        ],
        out_specs=out_spec,
        out_shape=out_shape,
    )(agg, w, b, cnt)


def _pad_idx(v, padval):
    v = v.reshape(NS, EPT)
    pad = jnp.full((NS, EPAD - EPT), padval, jnp.int32)
    return jnp.concatenate([v, pad], axis=1).reshape(NS, NCH, CHUNK)


def kernel(x, edge_index_r0, edge_index_r1, edge_index_r2, W1, b1, W2, b2):
    eis = (edge_index_r0, edge_index_r1, edge_index_r2)
    srcg = jnp.stack([_pad_idx(ei[0], 0) for ei in eis])
    srcd = jnp.stack([_pad_idx(ei[0], N) for ei in eis])
    dstd = jnp.stack([_pad_idx(ei[1], N) for ei in eis])

    cnt = _deg_call(srcd, dstd)
    hs1 = _prep_call(x, cnt)
    agg1 = _agg_call(hs1, srcg, dstd)
    hs2 = _layer_call(agg1, W1, b1, cnt, True)
    agg2 = _agg_call(hs2, srcg, dstd)
    out = _layer_call(agg2, W2, b2, cnt, False)
    return out


# speedup vs baseline: 1.1285x; 1.1285x over previous
import functools

import jax
import jax.numpy as jnp
from jax import lax
from jax.experimental import pallas as pl
from jax.experimental.pallas import tpu as pltpu
from jax.experimental.pallas import tpu_sc as plsc

N = 50000
D = 128
E = 160000
R = 3

NC = 2
NS = 16
LANES = 16

EPT = E // NS
CHUNK = 128
NCH = 80
EPAD = NCH * CHUNK

NP = 50176
TPT = NP // NS
ZCH = 784
WPT = N // NS
WCH = 125
ZBR = 64

CWA = 64


def _zero_fill(buf, rows, cols):
    z = jnp.zeros((LANES,), jnp.float32)

    def body(i, _):
        for k in range(cols // LANES):
            buf[i, pl.ds(k * LANES, LANES)] = z
        return 0

    lax.fori_loop(0, rows, body, 0)


def _deg_body(srcd, dstd, out, idxv, val, wb, acc):
    c = lax.axis_index("c")
    s = lax.axis_index("s")

    _zero_fill(wb, ZCH, 16)
    for q in range(TPT // ZCH):
        pltpu.sync_copy(wb, acc.at[pl.ds(s * TPT + q * ZCH, ZCH)])
    plsc.subcore_barrier()

    lane = lax.iota(jnp.int32, LANES)

    def run_task(t, idx_arr, r):
        oh = jnp.where(lane == t, 1.0, 0.0).astype(jnp.float32)

        def fill(i, _):
            val[i, pl.ds(0, LANES)] = oh
            return 0

        lax.fori_loop(0, CHUNK, fill, 0)
        pltpu.sync_copy(idx_arr.at[r, s], idxv)

        def body(j, _):
            pltpu.sync_copy(val, acc.at[idxv.at[j]], add=True)
            return 0

        lax.fori_loop(0, NCH, body, 0)

    for core in range(NC):
        @pl.when(c == core)
        def _():
            for t in range(3 * core, 3 * core + 3):
                r, which = divmod(t, 2)
                run_task(t, srcd if which == 0 else dstd, r)

    plsc.subcore_barrier()
    for q in range(TPT // ZCH):
        row = s * TPT + q * ZCH
        pltpu.sync_copy(acc.at[pl.ds(row, ZCH)], wb)
        pltpu.sync_copy(wb, out.at[c, pl.ds(row, ZCH)])


def _deg_call(srcd, dstd):
    fn = pl.kernel(
        _deg_body,
        out_type=jax.ShapeDtypeStruct((NC, NP, 16), jnp.float32),
        compiler_params=pltpu.CompilerParams(use_tc_tiling_on_sc=False),
        mesh=plsc.VectorSubcoreMesh(core_axis_name="c", subcore_axis_name="s"),
        scratch_types=[
            pltpu.VMEM((NCH, CHUNK), jnp.int32),
            pltpu.VMEM((CHUNK, 16), jnp.float32),
            pltpu.VMEM((ZCH, 16), jnp.float32),
            pltpu.VMEM_SHARED((NP, 16), jnp.float32),
        ],
    )
    return fn(srcd, dstd)


def _agg_body(hs12, srcg, dstd, zhbm, out, gidx, didx, gb0, gb1, zb, acc,
              sem0, sem1):
    c = lax.axis_index("c")
    s = lax.axis_index("s")

    pltpu.sync_copy(zhbm, zb)

    def run_pass(r):
        def zq(q, _):
            pltpu.sync_copy(zb, acc.at[pl.ds(s * TPT + q * ZBR, ZBR)])
            return 0

        lax.fori_loop(0, TPT // ZBR, zq, 0)
        plsc.subcore_barrier()

        pltpu.sync_copy(srcg.at[r, s], gidx)
        pltpu.sync_copy(dstd.at[r, s], didx)
        base = r * (2 * N) + c

        def xform(j, _):
            for k in range(CHUNK // LANES):
                v = gidx[j, pl.ds(k * LANES, LANES)]
                gidx[j, pl.ds(k * LANES, LANES)] = v * 2 + base
            return 0

        lax.fori_loop(0, NCH, xform, 0)

        pltpu.async_copy(hs12.at[gidx.at[0]], gb0, sem0)

        def pair(jj, _):
            j0 = 2 * jj
            pltpu.async_copy(hs12.at[gidx.at[j0 + 1]], gb1, sem1)
            pltpu.make_async_copy(hs12.at[gidx.at[j0]], gb0, sem0).wait()
            pltpu.sync_copy(gb0, acc.at[didx.at[j0]], add=True)

            @pl.when(jj + 1 < NCH // 2)
            def _():
                pltpu.async_copy(hs12.at[gidx.at[j0 + 2]], gb0, sem0)

            pltpu.make_async_copy(hs12.at[gidx.at[j0 + 1]], gb1, sem1).wait()
            pltpu.sync_copy(gb1, acc.at[didx.at[j0 + 1]], add=True)
            return 0

        lax.fori_loop(0, NCH // 2, pair, 0)
        plsc.subcore_barrier()

        pltpu.sync_copy(acc.at[pl.ds(s * WPT, WPT)],
                        out.at[r, pl.ds(s * WPT, WPT), pl.ds(c * CWA, CWA)])
        plsc.subcore_barrier()

    for r in range(R):
        run_pass(r)


def _agg_call(hs, srcg, dstd):
    hs12 = hs.reshape(R * N * 2, CWA)
    zhbm = jnp.zeros((ZBR, CWA), jnp.bfloat16)
    fn = pl.kernel(
        _agg_body,
        out_type=jax.ShapeDtypeStruct((R, N, D), jnp.bfloat16),
        compiler_params=pltpu.CompilerParams(use_tc_tiling_on_sc=False),
        mesh=plsc.VectorSubcoreMesh(core_axis_name="c", subcore_axis_name="s"),
        scratch_types=[
            pltpu.VMEM((NCH, CHUNK), jnp.int32),
            pltpu.VMEM((NCH, CHUNK), jnp.int32),
            pltpu.VMEM((CHUNK, CWA), jnp.bfloat16),
            pltpu.VMEM((CHUNK, CWA), jnp.bfloat16),
            pltpu.VMEM((ZBR, CWA), jnp.bfloat16),
            pltpu.VMEM_SHARED((NP, CWA), jnp.bfloat16),
            pltpu.SemaphoreType.DMA,
            pltpu.SemaphoreType.DMA,
        ],
    )
    return fn(hs12, srcg, dstd, zhbm)


BN = 2000
GRID = N // BN


def _rsqrt_deg(cnt_blk):
    c = cnt_blk[0] + cnt_blk[1]
    return lax.rsqrt(jnp.maximum(c, 1.0))


def _prep_body(x_ref, cnt_ref, hs_ref):
    d = _rsqrt_deg(cnt_ref[...])
    x = x_ref[...]
    for r in range(R):
        hs_ref[r] = (x * d[:, 2 * r:2 * r + 1]).astype(jnp.bfloat16)


def _prep_call(x, cnt):
    return pl.pallas_call(
        _prep_body,
        grid=(GRID,),
        in_specs=[
            pl.BlockSpec((BN, D), lambda i: (i, 0)),
            pl.BlockSpec((NC, BN, 16), lambda i: (0, i, 0)),
        ],
        out_specs=pl.BlockSpec((R, BN, D), lambda i: (0, i, 0)),
        out_shape=jax.ShapeDtypeStruct((R, N, D), jnp.bfloat16),
    )(x, cnt)


def _layer_body(agg_ref, w_ref, b_ref, cnt_ref, out_ref, *, relu_scale):
    d = _rsqrt_deg(cnt_ref[...])
    w = w_ref[...]
    b = b_ref[...]
    acc = (b[0] + b[1] + b[2])[None, :]
    for r in range(R):
        m = jnp.dot(agg_ref[r], w[r], preferred_element_type=jnp.float32)
        acc = acc + m * d[:, 2 * r + 1:2 * r + 2]
    if relu_scale:
        h = jnp.maximum(acc, 0.0)
        for r in range(R):
            out_ref[r] = (h * d[:, 2 * r:2 * r + 1]).astype(jnp.bfloat16)
    else:
        out_ref[...] = acc


def _layer_call(agg, w, b, cnt, relu_scale):
    h = w.shape[-1]
    if relu_scale:
        out_shape = jax.ShapeDtypeStruct((R, N, h), jnp.bfloat16)
        out_spec = pl.BlockSpec((R, BN, h), lambda i: (0, i, 0))
    else:
        out_shape = jax.ShapeDtypeStruct((N, h), jnp.float32)
        out_spec = pl.BlockSpec((BN, h), lambda i: (i, 0))
    return pl.pallas_call(
        functools.partial(_layer_body, relu_scale=relu_scale),
        grid=(GRID,),
        in_specs=[
            pl.BlockSpec((R, BN, D), lambda i: (0, i, 0)),
            pl.BlockSpec((R, D, h), lambda i: (0, 0, 0)),
            pl.BlockSpec((R, h), lambda i: (0, 0)),
            pl.BlockSpec((NC, BN, 16), lambda i: (0, i, 0)),
        ],
        out_specs=out_spec,
        out_shape=out_shape,
    )(agg, w, b, cnt)


def _pad_idx(v, padval):
    v = v.reshape(NS, EPT)
    pad = jnp.full((NS, EPAD - EPT), padval, jnp.int32)
    return jnp.concatenate([v, pad], axis=1).reshape(NS, NCH, CHUNK)


def kernel(x, edge_index_r0, edge_index_r1, edge_index_r2, W1, b1, W2, b2):
    eis = (edge_index_r0, edge_index_r1, edge_index_r2)
    srcg = jnp.stack([_pad_idx(ei[0], 0) for ei in eis])
    srcd = jnp.stack([_pad_idx(ei[0], N) for ei in eis])
    dstd = jnp.stack([_pad_idx(ei[1], N) for ei in eis])

    cnt = _deg_call(srcd, dstd)
    hs1 = _prep_call(x, cnt)
    agg1 = _agg_call(hs1, srcg, dstd)
    hs2 = _layer_call(agg1, W1, b1, cnt, True)
    agg2 = _agg_call(hs2, srcg, dstd)
    out = _layer_call(agg2, W2, b2, cnt, False)
    return out
